# Initial kernel scaffold; baseline (speedup 1.0000x reference)
import jax
import jax.numpy as jnp
from jax.experimental import pallas as pl


def kernel(inputs, edge_index, W1, Wl1, b1, W2, Wl2, b2):
    raise NotImplementedError("write your pallas kernel here")


# speedup vs baseline: 10.6866x; 10.6866x over previous
import functools

import jax
import jax.numpy as jnp
from jax import lax
from jax.experimental import pallas as pl
from jax.experimental.pallas import tpu as pltpu
from jax.experimental.pallas import tpu_sc as plsc

N = 10000
D = 128
E = 320000

NC = 2
NS = 16
NW = NC * NS
EPW = E // NW
K = 128
NCH = 80
EPWP = NCH * K
NBUF = 2
NGRP = NCH // NBUF
ROWS_A = 632
ROWS_L = N - (NS - 1) * ROWS_A

RB = 1000
GRID = N // RB


def _mm2_body(x_ref, w_ref, wl_ref, b_ref, s_ref, h_ref):
    x = x_ref[...]
    s_ref[...] = jnp.dot(x, w_ref[...], preferred_element_type=jnp.float32)
    h_ref[...] = jnp.dot(x, wl_ref[...], preferred_element_type=jnp.float32) + b_ref[...]


def _tc_mm(x, w, wl, b2d):
    return pl.pallas_call(
        _mm2_body,
        grid=(GRID,),
        in_specs=[
            pl.BlockSpec((RB, D), lambda i: (i, 0)),
            pl.BlockSpec((D, D), lambda i: (0, 0)),
            pl.BlockSpec((D, D), lambda i: (0, 0)),
            pl.BlockSpec((1, D), lambda i: (0, 0)),
        ],
        out_specs=[pl.BlockSpec((RB, D), lambda i: (i, 0))] * 2,
        out_shape=[jax.ShapeDtypeStruct((N, D), jnp.float32)] * 2,
    )(x, w, wl, b2d)


def _comb_mm_body(p_ref, hw_ref, w_ref, wl_ref, b_ref, s_ref, h_ref):
    x = p_ref[0] + p_ref[1] + hw_ref[...]
    s_ref[...] = jnp.dot(x, w_ref[...], preferred_element_type=jnp.float32)
    h_ref[...] = jnp.dot(x, wl_ref[...], preferred_element_type=jnp.float32) + b_ref[...]


def _tc_comb_mm(p, hw, w, wl, b2d):
    return pl.pallas_call(
        _comb_mm_body,
        grid=(GRID,),
        in_specs=[
            pl.BlockSpec((NC, RB, D), lambda i: (0, i, 0)),
            pl.BlockSpec((RB, D), lambda i: (i, 0)),
            pl.BlockSpec((D, D), lambda i: (0, 0)),
            pl.BlockSpec((D, D), lambda i: (0, 0)),
            pl.BlockSpec((1, D), lambda i: (0, 0)),
        ],
        out_specs=[pl.BlockSpec((RB, D), lambda i: (i, 0))] * 2,
        out_shape=[jax.ShapeDtypeStruct((N, D), jnp.float32)] * 2,
    )(p, hw, w, wl, b2d)


def _final_body(x0_ref, q_ref, hw_ref, o_ref):
    o_ref[...] = (x0_ref[...] + q_ref[0] + q_ref[1] + hw_ref[...]) * 0.5


def _tc_final(x0, q, hw):
    return pl.pallas_call(
        _final_body,
        grid=(GRID,),
        in_specs=[
            pl.BlockSpec((RB, D), lambda i: (i, 0)),
            pl.BlockSpec((NC, RB, D), lambda i: (0, i, 0)),
            pl.BlockSpec((RB, D), lambda i: (i, 0)),
        ],
        out_specs=pl.BlockSpec((RB, D), lambda i: (i, 0)),
        out_shape=jax.ShapeDtypeStruct((N, D), jnp.float32),
    )(x0, q, hw)


def _sc_body(sup, srci, dsti, zer, out, agg, si0, si1, di0, di1, rows,
             isem, gsem):
    cid = lax.axis_index("c")
    sid = lax.axis_index("s")
    wid = cid * NS + sid
    si = (si0, si1)
    di = (di0, di1)

    off = pl.multiple_of(sid * ROWS_A, 8)

    @pl.when(sid < NS - 1)
    def _():
        pltpu.sync_copy(zer.at[pl.ds(off, ROWS_A)], agg.at[pl.ds(off, ROWS_A)])

    @pl.when(sid == NS - 1)
    def _():
        pltpu.sync_copy(zer.at[pl.ds(off, ROWS_L)], agg.at[pl.ds(off, ROWS_L)])

    plsc.subcore_barrier()

    def start_idx(j, b):
        pltpu.async_copy(srci.at[wid * NCH + j, 0], si[b], isem.at[b])
        pltpu.async_copy(dsti.at[wid * NCH + j, 0], di[b], isem.at[b])

    def wait_idx(j, b):
        pltpu.make_async_copy(srci.at[wid * NCH + j, 0], si[b], isem.at[b]).wait()
        pltpu.make_async_copy(dsti.at[wid * NCH + j, 0], di[b], isem.at[b]).wait()

    def start_gather(b):
        pltpu.async_copy(sup.at[si[b]], rows.at[b], gsem.at[b])

    def wait_gather(b):
        pltpu.make_async_copy(sup.at[si[b]], rows.at[b], gsem.at[b]).wait()

    def scatter_add(b):
        pltpu.sync_copy(rows.at[b], agg.at[di[b]], add=True)

    start_idx(0, 0)
    start_idx(1, 1)
    wait_idx(0, 0)
    start_gather(0)

    @pl.loop(0, NGRP)
    def _grp(g):
        j0 = g * NBUF
        for b in range(NBUF):
            j = j0 + b
            b1 = (b + 1) % NBUF

            @pl.when(j + 1 < NCH)
            def _():
                wait_idx(j + 1, b1)
                start_gather(b1)

            wait_gather(b)
            scatter_add(b)

            @pl.when(j + 2 < NCH)
            def _():
                start_idx(j + 2, b)

    plsc.subcore_barrier()

    @pl.when(sid < NS - 1)
    def _():
        pltpu.sync_copy(agg.at[pl.ds(off, ROWS_A)],
                        out.at[cid, pl.ds(off, ROWS_A)])

    @pl.when(sid == NS - 1)
    def _():
        pltpu.sync_copy(agg.at[pl.ds(off, ROWS_L)],
                        out.at[cid, pl.ds(off, ROWS_L)])


_sc_seg = functools.partial(
    pl.kernel,
    out_type=jax.ShapeDtypeStruct((NC, N, D), jnp.float32),
    mesh=plsc.VectorSubcoreMesh(core_axis_name="c", subcore_axis_name="s"),
    scratch_types=[
        pltpu.VMEM_SHARED((N + 8, D), jnp.float32),
        pltpu.VMEM((K,), jnp.int32),
        pltpu.VMEM((K,), jnp.int32),
        pltpu.VMEM((K,), jnp.int32),
        pltpu.VMEM((K,), jnp.int32),
        pltpu.VMEM((NBUF, K, D), jnp.float32),
        pltpu.SemaphoreType.DMA((NBUF,)),
        pltpu.SemaphoreType.DMA((NBUF,)),
    ],
)(_sc_body)


def kernel(inputs, edge_index, W1, Wl1, b1, W2, Wl2, b2):
    x0 = inputs
    npad = EPWP - EPW
    src_pad = jnp.broadcast_to((jnp.arange(npad, dtype=jnp.int32) * 37) % N,
                               (NW, npad))
    dst_pad = jnp.broadcast_to(N + (jnp.arange(npad, dtype=jnp.int32) % 8),
                               (NW, npad))
    src = jnp.concatenate([edge_index[0].reshape(NW, EPW), src_pad], axis=1)
    dst = jnp.concatenate([edge_index[1].reshape(NW, EPW), dst_pad], axis=1)
    src = src.reshape(NW * NCH, 1, K)
    dst = dst.reshape(NW * NCH, 1, K)
    zer = jnp.zeros((N + 8, D), jnp.float32)
    b1r = b1.reshape(1, D)
    b2r = b2.reshape(1, D)

    s1, h1 = _tc_mm(x0, W1, Wl1, b1r)
    p = _sc_seg(s1, src, dst, zer)
    s2, h2 = _tc_comb_mm(p, h1, W2, Wl2, b2r)
    q = _sc_seg(s2, src, dst, zer)
    return _tc_final(x0, q, h2)
